# baseline (speedup 1.0000x reference)
import functools

import jax
import jax.numpy as jnp
from jax import lax
from jax.experimental import pallas as pl
from jax.experimental.pallas import tpu as pltpu
from jax.experimental.pallas import tpu_sc as plsc

_ROWS_PER_BLOCK = 8192
_LANES = 16


def _argmax_body(x_ref, w_ref, idx_ref):
    xb = x_ref[...]
    m = jnp.max(xb, axis=1, keepdims=True)
    eqf = (xb == m).astype(jnp.float32)
    p = lax.dot_general(
        eqf, w_ref[...], (((1,), (0,)), ((), ())),
        preferred_element_type=jnp.float32,
    )
    ebits = lax.shift_right_logical(
        lax.bitcast_convert_type(p, jnp.int32), 23
    )
    c = lax.shift_right_logical((127 + _W_BIAS) - ebits, 1)
    idx = jnp.where(
        p[:, 0] > 0.0, c[:, 0],
        jnp.where(p[:, 1] > 0.0, c[:, 1] + 64,
                  jnp.where(p[:, 2] > 0.0, c[:, 2] + 128, c[:, 3] + 192)),
    )
    idx_ref[...] = idx


_W_BIAS = 63
_W_GROUPS = 4


def _index_weights(ncol):
    import numpy as np
    gsz = ncol // _W_GROUPS
    w = np.zeros((ncol, _W_GROUPS), dtype=np.float32)
    pw = np.exp2(_W_BIAS - 2.0 * np.arange(gsz, dtype=np.float64))
    for g in range(_W_GROUPS):
        w[g * gsz:(g + 1) * gsz, g] = pw.astype(np.float32)
    return jnp.asarray(w)


def _tc_argmax(x):
    n, c = x.shape
    r = _ROWS_PER_BLOCK
    w = _index_weights(c)
    return pl.pallas_call(
        _argmax_body,
        grid=(n // r,),
        in_specs=[
            pl.BlockSpec((r, c), lambda i: (i, 0)),
            pl.BlockSpec((c, _W_GROUPS), lambda i: (0, 0)),
        ],
        out_specs=pl.BlockSpec((r,), lambda i: (i,)),
        out_shape=jax.ShapeDtypeStruct((n,), jnp.int32),
    )(x, w)


def _sc_lookup(table_flat, idx):
    n = idx.shape[0]
    info = plsc.get_sparse_core_info()
    nw = info.num_cores * info.num_subcores
    bpw = n // nw
    mesh = plsc.VectorSubcoreMesh(core_axis_name="c", subcore_axis_name="s")

    @functools.partial(
        pl.kernel,
        mesh=mesh,
        out_type=jax.ShapeDtypeStruct((2 * n,), jnp.float32),
        scratch_types=[
            pltpu.VMEM((table_flat.shape[0],), jnp.float32),
            pltpu.VMEM((bpw,), jnp.int32),
            pltpu.VMEM((2 * bpw,), jnp.float32),
        ],
        compiler_params=pltpu.CompilerParams(needs_layout_passes=False),
    )
    def _k(table_hbm, idx_hbm, out_hbm, tbl_v, idx_v, out_v):
        wid = lax.axis_index("s") * info.num_cores + lax.axis_index("c")
        base = wid * bpw
        pltpu.sync_copy(table_hbm, tbl_v)
        pltpu.sync_copy(idx_hbm.at[pl.ds(base, bpw)], idx_v)

        def body(i, carry):
            off = pl.multiple_of(i * _LANES, _LANES)
            iv = idx_v[pl.ds(off, _LANES)]
            first = plsc.load_gather(tbl_v, [iv * 2])
            second = plsc.load_gather(tbl_v, [iv * 2 + 1])
            pos = (lax.iota(jnp.int32, _LANES) + off) * 2
            plsc.store_scatter(out_v, [pos], first)
            plsc.store_scatter(out_v, [pos + 1], second)
            return carry

        lax.fori_loop(0, bpw // _LANES, body, 0)
        pltpu.sync_copy(out_v, out_hbm.at[pl.ds(2 * base, 2 * bpw)])

    return _k(table_flat, idx)


def kernel(x, mapping):
    idx = _tc_argmax(x)
    flat = _sc_lookup(mapping.reshape(-1), idx)
    return flat.reshape(x.shape[0], 2)
